# baseline (speedup 1.0000x reference)
import functools

import jax
import jax.numpy as jnp
from jax import lax
from jax.experimental import pallas as pl
from jax.experimental.pallas import tpu as pltpu
from jax.experimental.pallas import tpu_sc as plsc

N = 10000
E = 320000
D = 128
NC = 2
NS = 16
NW = NC * NS
C = 128
CH = 80
G = 16
NG = CH // G
EPW = C * CH
EP = EPW * NW
RPT = 640
NPAD = RPT * NS

_mesh = plsc.VectorSubcoreMesh(core_axis_name="c", subcore_axis_name="s")


@functools.partial(
    pl.kernel,
    mesh=_mesh,
    out_type=[
        jax.ShapeDtypeStruct((NW, RPT, D), jnp.float32),
        jax.ShapeDtypeStruct((NC, NPAD), jnp.float32),
    ],
    scratch_types=[
        pltpu.VMEM((2, G, C), jnp.int32),
        pltpu.VMEM((2, G, C), jnp.int32),
        pltpu.VMEM((C, D), jnp.float32),
        pltpu.VMEM((C, D), jnp.float32),
        pltpu.VMEM((C,), jnp.float32),
        pltpu.VMEM((RPT,), jnp.float32),
        pltpu.VMEM_SHARED((NPAD, D), jnp.float32),
        pltpu.VMEM_SHARED((NPAD,), jnp.float32),
        pltpu.SemaphoreType.DMA,
        pltpu.SemaphoreType.DMA,
        pltpu.SemaphoreType.DMA,
        pltpu.SemaphoreType.DMA,
        pltpu.SemaphoreType.DMA,
        pltpu.SemaphoreType.DMA,
        pltpu.SemaphoreType.DMA,
    ],
)
def _sc_scatter(x_hbm, src_hbm, dst_hbm, tails_hbm, taild_hbm, sums_hbm,
                deg_hbm, srcb, dstb, rowb0, rowb1, onesb, zb, shared,
                shared_deg, sem0, sem1, semis, semid, semsc0, semsc1,
                semdg):
    rbufs = (rowb0, rowb1)
    sems = (sem0, sem1)
    semsc = (semsc0, semsc1)
    rowb = rowb0
    c = lax.axis_index("c")
    s = lax.axis_index("s")
    wid = s * NC + c
    oid = c * NS + s
    base = s * RPT

    z16 = jnp.zeros((16,), jnp.float32)
    ones16 = jnp.ones((16,), jnp.float32)

    for g in range(C // 16):
        onesb[pl.ds(g * 16, 16)] = ones16

    def zero_zb(i, carry):
        zb[pl.ds(i * 16, 16)] = z16
        return carry
    lax.fori_loop(0, RPT // 16, zero_zb, 0)

    def zero_row(r, carry):
        for g in range(D // 16):
            rowb[r, pl.ds(g * 16, 16)] = z16
        return carry
    lax.fori_loop(0, C, zero_row, 0)

    for t in range(RPT // C):
        pltpu.sync_copy(rowb, shared.at[pl.ds(base + t * C, C)])
    pltpu.sync_copy(zb, shared_deg.at[pl.ds(base, RPT)])

    def stage_idx(goff, bufidx):
        @pl.when(wid < NW - 1)
        def _():
            pltpu.async_copy(src_hbm.at[pl.ds(wid * CH + goff, G)],
                             srcb.at[bufidx], semis)
            pltpu.async_copy(dst_hbm.at[pl.ds(wid * CH + goff, G)],
                             dstb.at[bufidx], semid)

        @pl.when(wid == NW - 1)
        def _():
            pltpu.async_copy(tails_hbm.at[pl.ds(goff, G)],
                             srcb.at[bufidx], semis)
            pltpu.async_copy(taild_hbm.at[pl.ds(goff, G)],
                             dstb.at[bufidx], semid)

    def drain_idx():
        pltpu.make_async_copy(tails_hbm.at[pl.ds(0, G)],
                              srcb.at[0], semis).wait()
        pltpu.make_async_copy(taild_hbm.at[pl.ds(0, G)],
                              dstb.at[0], semid).wait()

    stage_idx(0, 0)
    drain_idx()
    pltpu.async_copy(x_hbm.at[dstb.at[0, 0]], rbufs[0], sems[0])

    plsc.subcore_barrier()

    def wait_row_scatter(b):
        pltpu.make_async_copy(rbufs[b], shared.at[pl.ds(0, C)],
                              semsc[b]).wait()

    for g in range(NG):
        gb = g % 2
        if g + 1 < NG:
            stage_idx((g + 1) * G, 1 - gb)

        def chunk_pair(l2, carry, gb=gb):
            for b in range(2):
                l = l2 * 2 + b
                nxt = l + 1
                if b == 0:
                    @pl.when(l2 >= 1)
                    def _():
                        wait_row_scatter(1)
                    pltpu.async_copy(x_hbm.at[dstb.at[gb, nxt]],
                                     rbufs[1], sems[1])
                else:
                    wait_row_scatter(0)

                    @pl.when(nxt < G)
                    def _():
                        pltpu.async_copy(x_hbm.at[dstb.at[gb, nxt]],
                                         rbufs[0], sems[0])
                pltpu.make_async_copy(x_hbm.at[dstb.at[gb, l]], rbufs[b],
                                      sems[b]).wait()
                pltpu.async_copy(rbufs[b], shared.at[srcb.at[gb, l]],
                                 semsc[b], add=True)
                pltpu.async_copy(onesb, shared_deg.at[srcb.at[gb, l]],
                                 semdg, add=True)
            return carry
        lax.fori_loop(0, G // 2, chunk_pair, 0)

        wait_row_scatter(1)

        def drain_deg(i, carry):
            pltpu.make_async_copy(onesb, shared_deg.at[pl.ds(0, C)],
                                  semdg).wait()
            return carry
        lax.fori_loop(0, G, drain_deg, 0)

        if g + 1 < NG:
            drain_idx()
            pltpu.async_copy(x_hbm.at[dstb.at[1 - gb, 0]], rbufs[0],
                             sems[0])

    plsc.subcore_barrier()

    pltpu.sync_copy(shared.at[pl.ds(base, RPT)], sums_hbm.at[oid])
    pltpu.sync_copy(shared_deg.at[pl.ds(base, RPT)],
                    deg_hbm.at[c, pl.ds(base, RPT)])


TCB = 2
BR = TCB * RPT


def _tc_body(x_ref, p0_ref, p1_ref, dg_ref, ws_ref, bs_ref, wn_ref, bn_ref,
             o_ref):
    x = x_ref[...]
    hs = lax.dot_general(x, ws_ref[...], (((1,), (1,)), ((), ())),
                         preferred_element_type=jnp.float32) + bs_ref[...]
    p = (p0_ref[...] + p1_ref[...]).reshape(BR, D)
    dg = dg_ref[...]
    d = (dg[0] + dg[1] + 1e-9).reshape(BR, 1)
    pn = p / d
    hn = lax.dot_general(pn, wn_ref[...], (((1,), (1,)), ((), ())),
                         preferred_element_type=jnp.float32) + bn_ref[...]
    o_ref[...] = jnp.tanh(jnp.concatenate([hs, hn], axis=1))


def kernel(X, indices, W_self, b_self, W_neigh, b_neigh):
    idx = indices.astype(jnp.int32)
    pad = EP - E
    pad_src = N + (jnp.arange(pad, dtype=jnp.int32) % (NPAD - N))
    pad_dst = jnp.arange(pad, dtype=jnp.int32) % N
    tail_s = jnp.concatenate(
        [idx[0, (NW - 1) * EPW:], pad_src]).reshape(CH, C)
    tail_d = jnp.concatenate(
        [idx[1, (NW - 1) * EPW:], pad_dst]).reshape(CH, C)

    sums, degs = _sc_scatter(X, idx[0].reshape(E // C, C),
                             idx[1].reshape(E // C, C), tail_s, tail_d)

    out = pl.pallas_call(
        _tc_body,
        grid=(NPAD // BR,),
        in_specs=[
            pl.BlockSpec((BR, D), lambda i: (i, 0)),
            pl.BlockSpec((TCB, RPT, D), lambda i: (i, 0, 0)),
            pl.BlockSpec((TCB, RPT, D), lambda i: (NS // TCB + i, 0, 0)),
            pl.BlockSpec((NC, BR), lambda i: (0, i)),
            pl.BlockSpec((D, D), lambda i: (0, 0)),
            pl.BlockSpec((1, D), lambda i: (0, 0)),
            pl.BlockSpec((D, D), lambda i: (0, 0)),
            pl.BlockSpec((1, D), lambda i: (0, 0)),
        ],
        out_specs=pl.BlockSpec((BR, 2 * D), lambda i: (i, 0)),
        out_shape=jax.ShapeDtypeStruct((N, 2 * D), jnp.float32),
    )(X, sums, sums, degs, W_self, b_self.reshape(1, D), W_neigh,
      b_neigh.reshape(1, D))
    return out


# speedup vs baseline: 1.0728x; 1.0728x over previous
import functools

import jax
import jax.numpy as jnp
from jax import lax
from jax.experimental import pallas as pl
from jax.experimental.pallas import tpu as pltpu
from jax.experimental.pallas import tpu_sc as plsc

N = 10000
E = 320000
D = 128
NC = 2
NS = 16
NW = NC * NS
C = 128
CH = 80
G = 16
NG = CH // G
EPW = C * CH
EP = EPW * NW
RPT = 640
NPAD = RPT * NS

_mesh = plsc.VectorSubcoreMesh(core_axis_name="c", subcore_axis_name="s")


@functools.partial(
    pl.kernel,
    mesh=_mesh,
    out_type=[
        jax.ShapeDtypeStruct((NW, RPT, D), jnp.float32),
        jax.ShapeDtypeStruct((NC, NPAD), jnp.float32),
    ],
    scratch_types=[
        pltpu.VMEM((2, G, C), jnp.int32),
        pltpu.VMEM((2, G, C), jnp.int32),
        pltpu.VMEM((C, D), jnp.float32),
        pltpu.VMEM((C, D), jnp.float32),
        pltpu.VMEM((C,), jnp.float32),
        pltpu.VMEM((RPT,), jnp.float32),
        pltpu.VMEM_SHARED((NPAD, D), jnp.float32),
        pltpu.VMEM_SHARED((NPAD,), jnp.float32),
        pltpu.SemaphoreType.DMA,
        pltpu.SemaphoreType.DMA,
        pltpu.SemaphoreType.DMA,
        pltpu.SemaphoreType.DMA,
        pltpu.SemaphoreType.DMA,
        pltpu.SemaphoreType.DMA,
        pltpu.SemaphoreType.DMA,
    ],
)
def _sc_scatter(x_hbm, idx_hbm, tail_hbm, sums_hbm, deg_hbm,
                srcb, dstb, rowb0, rowb1, onesb, zb, shared, shared_deg,
                sem0, sem1, semis, semid, semsc0, semsc1, semdg):
    rbufs = (rowb0, rowb1)
    sems = (sem0, sem1)
    semsc = (semsc0, semsc1)
    rowb = rowb0
    c = lax.axis_index("c")
    s = lax.axis_index("s")
    wid = s * NC + c
    oid = c * NS + s
    base = s * RPT

    z16 = jnp.zeros((16,), jnp.float32)
    ones16 = jnp.ones((16,), jnp.float32)

    for g in range(C // 16):
        onesb[pl.ds(g * 16, 16)] = ones16

    def zero_zb(i, carry):
        zb[pl.ds(i * 16, 16)] = z16
        return carry
    lax.fori_loop(0, RPT // 16, zero_zb, 0)

    def zero_row(r, carry):
        for g in range(D // 16):
            rowb[r, pl.ds(g * 16, 16)] = z16
        return carry
    lax.fori_loop(0, C, zero_row, 0)

    for t in range(RPT // C):
        pltpu.sync_copy(rowb, shared.at[pl.ds(base + t * C, C)])
    pltpu.sync_copy(zb, shared_deg.at[pl.ds(base, RPT)])

    def stage_idx(goff, bufidx):
        @pl.when(wid < NW - 1)
        def _():
            pltpu.async_copy(idx_hbm.at[0, pl.ds(wid * CH + goff, G)],
                             srcb.at[bufidx], semis)
            pltpu.async_copy(idx_hbm.at[1, pl.ds(wid * CH + goff, G)],
                             dstb.at[bufidx], semid)

        @pl.when(wid == NW - 1)
        def _():
            pltpu.async_copy(tail_hbm.at[0, pl.ds(goff, G)],
                             srcb.at[bufidx], semis)
            pltpu.async_copy(tail_hbm.at[1, pl.ds(goff, G)],
                             dstb.at[bufidx], semid)

    def drain_idx():
        pltpu.make_async_copy(tail_hbm.at[0, pl.ds(0, G)],
                              srcb.at[0], semis).wait()
        pltpu.make_async_copy(tail_hbm.at[1, pl.ds(0, G)],
                              dstb.at[0], semid).wait()

    stage_idx(0, 0)
    drain_idx()
    pltpu.async_copy(x_hbm.at[dstb.at[0, 0]], rbufs[0], sems[0])

    plsc.subcore_barrier()

    def wait_row_scatter(b):
        pltpu.make_async_copy(rbufs[b], shared.at[pl.ds(0, C)],
                              semsc[b]).wait()

    for g in range(NG):
        gb = g % 2
        if g + 1 < NG:
            stage_idx((g + 1) * G, 1 - gb)

        def chunk_pair(l2, carry, gb=gb):
            for b in range(2):
                l = l2 * 2 + b
                nxt = l + 1
                if b == 0:
                    @pl.when(l2 >= 1)
                    def _():
                        wait_row_scatter(1)
                    pltpu.async_copy(x_hbm.at[dstb.at[gb, nxt]],
                                     rbufs[1], sems[1])
                else:
                    wait_row_scatter(0)

                    @pl.when(nxt < G)
                    def _():
                        pltpu.async_copy(x_hbm.at[dstb.at[gb, nxt]],
                                         rbufs[0], sems[0])
                pltpu.make_async_copy(x_hbm.at[dstb.at[gb, l]], rbufs[b],
                                      sems[b]).wait()
                pltpu.async_copy(rbufs[b], shared.at[srcb.at[gb, l]],
                                 semsc[b], add=True)
                pltpu.async_copy(onesb, shared_deg.at[srcb.at[gb, l]],
                                 semdg, add=True)
            return carry
        lax.fori_loop(0, G // 2, chunk_pair, 0)

        wait_row_scatter(1)

        def drain_deg(i, carry):
            pltpu.make_async_copy(onesb, shared_deg.at[pl.ds(0, C)],
                                  semdg).wait()
            return carry
        lax.fori_loop(0, G, drain_deg, 0)

        if g + 1 < NG:
            drain_idx()
            pltpu.async_copy(x_hbm.at[dstb.at[1 - gb, 0]], rbufs[0],
                             sems[0])

    plsc.subcore_barrier()

    pltpu.sync_copy(shared.at[pl.ds(base, RPT)], sums_hbm.at[oid])
    pltpu.sync_copy(shared_deg.at[pl.ds(base, RPT)],
                    deg_hbm.at[c, pl.ds(base, RPT)])


TCB = 2
BR = TCB * RPT


def _tc_body(x_ref, p0_ref, p1_ref, dg_ref, ws_ref, bs_ref, wn_ref, bn_ref,
             o_ref):
    x = x_ref[...]
    hs = lax.dot_general(x, ws_ref[...], (((1,), (1,)), ((), ())),
                         preferred_element_type=jnp.float32) + bs_ref[...]
    p = (p0_ref[...] + p1_ref[...]).reshape(BR, D)
    dg = dg_ref[...]
    d = (dg[0] + dg[1] + 1e-9).reshape(BR, 1)
    pn = p / d
    hn = lax.dot_general(pn, wn_ref[...], (((1,), (1,)), ((), ())),
                         preferred_element_type=jnp.float32) + bn_ref[...]
    o_ref[...] = jnp.tanh(jnp.concatenate([hs, hn], axis=1))


def kernel(X, indices, W_self, b_self, W_neigh, b_neigh):
    idx = indices.astype(jnp.int32)
    pad = EP - E
    pad_src = N + (jnp.arange(pad, dtype=jnp.int32) % (NPAD - N))
    pad_dst = jnp.arange(pad, dtype=jnp.int32) % N
    tail = jnp.concatenate(
        [idx[:, (NW - 1) * EPW:], jnp.stack([pad_src, pad_dst])],
        axis=1).reshape(2, CH, C)

    sums, degs = _sc_scatter(X, idx.reshape(2, E // C, C), tail)

    out = pl.pallas_call(
        _tc_body,
        grid=(NPAD // BR,),
        in_specs=[
            pl.BlockSpec((BR, D), lambda i: (i, 0)),
            pl.BlockSpec((TCB, RPT, D), lambda i: (i, 0, 0)),
            pl.BlockSpec((TCB, RPT, D), lambda i: (NS // TCB + i, 0, 0)),
            pl.BlockSpec((NC, BR), lambda i: (0, i)),
            pl.BlockSpec((D, D), lambda i: (0, 0)),
            pl.BlockSpec((1, D), lambda i: (0, 0)),
            pl.BlockSpec((D, D), lambda i: (0, 0)),
            pl.BlockSpec((1, D), lambda i: (0, 0)),
        ],
        out_specs=pl.BlockSpec((BR, 2 * D), lambda i: (i, 0)),
        out_shape=jax.ShapeDtypeStruct((N, 2 * D), jnp.float32),
    )(X, sums, sums, degs, W_self, b_self.reshape(1, D), W_neigh,
      b_neigh.reshape(1, D))
    return out
